# Initial kernel scaffold; baseline (speedup 1.0000x reference)
import jax
import jax.numpy as jnp
from jax.experimental import pallas as pl


def kernel(x, ei_sends, ei_receives, Wp, bp, Wl, bl, Wr, br, att, bias, gamma, beta, Wc, bc):
    raise NotImplementedError("write your pallas kernel here")


# speedup vs baseline: 1.0933x; 1.0933x over previous
import functools

import jax
import jax.numpy as jnp
from jax import lax
from jax.experimental import pallas as pl
from jax.experimental.pallas import tpu as pltpu

_N = 50000
_D = 128
_H = 4
_P = 32
_E = 400000
_L = 3
_BLK = 1000


def _pre_body(x_ref, w_ref, b_ref, o_ref):
    h = jnp.dot(x_ref[...], w_ref[...], preferred_element_type=jnp.float32)
    h = h + b_ref[...]
    o_ref[...] = jnp.where(h > 0, h, jnp.exp(h) - 1.0)


def _pre(x, Wp, bp):
    return pl.pallas_call(
        _pre_body,
        grid=(_N // _BLK,),
        in_specs=[
            pl.BlockSpec((_BLK, _D), lambda i: (i, 0)),
            pl.BlockSpec((_D, _D), lambda i: (0, 0)),
            pl.BlockSpec((1, _D), lambda i: (0, 0)),
        ],
        out_specs=pl.BlockSpec((_BLK, _D), lambda i: (i, 0)),
        out_shape=jax.ShapeDtypeStruct((_N, _D), jnp.float32),
    )(x, Wp, bp)


def _proj_body(v_ref, sc_ref, sh_ref, w_ref, b_ref, o_ref):
    hblk = v_ref[...] * sc_ref[...] + sh_ref[...]
    w = w_ref[...]
    b = b_ref[...]
    for t in range(4):
        o_ref[t] = jnp.dot(hblk, w[t], preferred_element_type=jnp.float32) + b[t]


def _proj(v, scale, shift, W4, b4):
    return pl.pallas_call(
        _proj_body,
        grid=(_N // _BLK,),
        in_specs=[
            pl.BlockSpec((_BLK, _D), lambda i: (i, 0)),
            pl.BlockSpec((1, _D), lambda i: (0, 0)),
            pl.BlockSpec((1, _D), lambda i: (0, 0)),
            pl.BlockSpec((4, _D, _D), lambda i: (0, 0, 0)),
            pl.BlockSpec((4, 1, _D), lambda i: (0, 0, 0)),
        ],
        out_specs=pl.BlockSpec((4, _BLK, _D), lambda i: (0, i, 0)),
        out_shape=jax.ShapeDtypeStruct((4, _N, _D), jnp.float32),
    )(v, scale, shift, W4, b4)


def _post_body(a0_ref, a1_ref, b_ref, v_ref, st_ref):
    s = a0_ref[...] + a1_ref[...]
    o = jnp.concatenate([s[0], s[1], s[2], s[3]], axis=-1) + b_ref[...]
    v = jnp.where(o > 0, o, jnp.exp(o) - 1.0)
    v_ref[...] = v

    @pl.when(pl.program_id(0) == 0)
    def _():
        st_ref[...] = jnp.zeros_like(st_ref)

    st_ref[0:1, :] += jnp.sum(v, axis=0, keepdims=True)
    st_ref[1:2, :] += jnp.sum(v * v, axis=0, keepdims=True)


def _post(agg0, agg1, bsum):
    return pl.pallas_call(
        _post_body,
        grid=(_N // _BLK,),
        in_specs=[
            pl.BlockSpec((_H, _BLK, _P), lambda i: (0, i, 0)),
            pl.BlockSpec((_H, _BLK, _P), lambda i: (0, i, 0)),
            pl.BlockSpec((1, _D), lambda i: (0, 0)),
        ],
        out_specs=[
            pl.BlockSpec((_BLK, _D), lambda i: (i, 0)),
            pl.BlockSpec((2, _D), lambda i: (0, 0)),
        ],
        out_shape=[
            jax.ShapeDtypeStruct((_N, _D), jnp.float32),
            jax.ShapeDtypeStruct((2, _D), jnp.float32),
        ],
        compiler_params=pltpu.CompilerParams(
            dimension_semantics=("arbitrary",)),
    )(agg0, agg1, bsum)


def _fin_body(v_ref, sc_ref, sh_ref, w_ref, o_ref):
    hblk = v_ref[...] * sc_ref[...] + sh_ref[...]
    o_ref[...] = jnp.dot(hblk, w_ref[...], preferred_element_type=jnp.float32)


def _fin(v, scale, shift, Wc_pad):
    return pl.pallas_call(
        _fin_body,
        grid=(_N // _BLK,),
        in_specs=[
            pl.BlockSpec((_BLK, _D), lambda i: (i, 0)),
            pl.BlockSpec((1, _D), lambda i: (0, 0)),
            pl.BlockSpec((1, _D), lambda i: (0, 0)),
            pl.BlockSpec((_D, _D), lambda i: (0, 0)),
        ],
        out_specs=pl.BlockSpec((_BLK, _D), lambda i: (i, 0)),
        out_shape=jax.ShapeDtypeStruct((_N, _D), jnp.float32),
    )(v, scale, shift, Wc_pad)


def _edge(xl, xr, ei, att):
    src, dst = ei[0], ei[1]
    e = xl[src].reshape(_E, _H, _P) + xr[dst].reshape(_E, _H, _P)
    e = jnp.where(e > 0, e, 0.2 * e)
    logits = (e * att[None]).sum(-1)
    ex = jnp.exp(logits)
    den = jax.ops.segment_sum(ex, dst, num_segments=_N)
    alpha = ex * (1.0 / (den + 1e-16))[dst]
    msg = xl[src].reshape(_E, _H, _P) * alpha[:, :, None]
    out = jax.ops.segment_sum(msg, dst, num_segments=_N)
    return out.transpose(1, 0, 2)


def kernel(x, ei_sends, ei_receives, Wp, bp, Wl, bl, Wr, br, att, bias,
           gamma, beta, Wc, bc):
    v = _pre(x, Wp, bp.reshape(1, _D))
    scale = jnp.ones((1, _D), jnp.float32)
    shift = jnp.zeros((1, _D), jnp.float32)
    eis = (ei_sends, ei_receives)
    for l in range(_L):
        W4 = jnp.stack([Wl[l, 0], Wr[l, 0], Wl[l, 1], Wr[l, 1]])
        b4 = jnp.stack([bl[l, 0], br[l, 0], bl[l, 1], br[l, 1]]).reshape(4, 1, _D)
        xs = _proj(v, scale, shift, W4, b4)
        agg0 = _edge(xs[0], xs[1], eis[0], att[l, 0])
        agg1 = _edge(xs[2], xs[3], eis[1], att[l, 1])
        bsum = (bias[l, 0] + bias[l, 1]).reshape(1, _D)
        v, st = _post(agg0, agg1, bsum)
        mu = st[0] / _N
        var = st[1] / _N - mu * mu
        inv = lax.rsqrt(var + 1e-5)
        scale = (inv * gamma[l]).reshape(1, _D)
        shift = (beta[l] - mu * inv * gamma[l]).reshape(1, _D)
    Wc_pad = jnp.zeros((_D, _D), jnp.float32).at[:, :2].set(Wc)
    y = _fin(v, scale, shift, Wc_pad)
    return y[:, :2] + bc
